# baseline (speedup 1.0000x reference)
import functools

import jax
import jax.numpy as jnp
from jax import lax
from jax.experimental import pallas as pl
from jax.experimental.pallas import tpu as pltpu
from jax.experimental.pallas import tpu_sc as plsc

N_ATOMS = 10000
N_BONDS = 320000
MAX_NB = 32
ATOM_FDIM = 128
BOND_FDIM = 144
HIDDEN = 256
DEPTH = 4

NC = 2
NS = 16
NW = NC * NS
LANES = 16

AT_PER_W = 320
NPAD = NW * AT_PER_W
A_IDX_ROW = 128
A_CHUNK_ATOMS = A_IDX_ROW // MAX_NB
A_CHUNKS = AT_PER_W // A_CHUNK_ATOMS

E_PER_W = N_BONDS // NW
B_CHUNK = 80
B_CHUNKS = E_PER_W // B_CHUNK

_sc_mesh = plsc.VectorSubcoreMesh(
    core_axis_name="c", subcore_axis_name="s", num_cores=NC, num_subcores=NS)


def _worker_id():
  return lax.axis_index("s") * NC + lax.axis_index("c")


@functools.partial(
    pl.kernel,
    out_type=jax.ShapeDtypeStruct((NPAD, HIDDEN), jnp.float32),
    mesh=_sc_mesh,
    scratch_types=[
        pltpu.VMEM((A_CHUNKS, A_IDX_ROW), jnp.int32),
        pltpu.VMEM((A_IDX_ROW, HIDDEN), jnp.float32),
        pltpu.VMEM((A_IDX_ROW, HIDDEN), jnp.float32),
        pltpu.VMEM((2 * A_CHUNK_ATOMS, HIDDEN), jnp.float32),
        pltpu.SemaphoreType.DMA,
        pltpu.SemaphoreType.DMA,
    ],
)
def _sc_gather_sum(msg_hbm, a2b_hbm, out_hbm, idx_v, rows0, rows1, sum_v,
                   sem0, sem1):
  wid = _worker_id()
  base_atom = wid * AT_PER_W
  pltpu.sync_copy(a2b_hbm.at[wid], idx_v)

  def issue(ci, buf, sem):
    pltpu.async_copy(msg_hbm.at[idx_v.at[ci]], buf, sem)

  def drain(ci, buf, sem):
    pltpu.make_async_copy(msg_hbm.at[idx_v.at[ci]], buf, sem).wait()

  def reduce_into(buf, atom_off):
    def red(a, _):
      base = a * MAX_NB
      for j in range(HIDDEN // LANES):
        col = j * LANES
        vals = [buf[base + r, pl.ds(col, LANES)] for r in range(MAX_NB)]
        while len(vals) > 1:
          vals = [vals[i] + vals[i + 1] for i in range(0, len(vals), 2)]
        sum_v[atom_off + a, pl.ds(col, LANES)] = vals[0]
      return 0

    lax.fori_loop(0, A_CHUNK_ATOMS, red, 0)

  n2 = A_CHUNKS // 2
  issue(0, rows0, sem0)
  issue(1, rows1, sem1)

  def pair(c2, _):
    c0 = 2 * c2
    drain(c0, rows0, sem0)
    reduce_into(rows0, 0)

    @pl.when(c0 + 2 < A_CHUNKS)
    def _():
      issue(c0 + 2, rows0, sem0)

    drain(c0 + 1, rows1, sem1)
    reduce_into(rows1, A_CHUNK_ATOMS)

    @pl.when(c0 + 3 < A_CHUNKS)
    def _():
      issue(c0 + 3, rows1, sem1)

    pltpu.sync_copy(
        sum_v,
        out_hbm.at[pl.ds(base_atom + c2 * 2 * A_CHUNK_ATOMS,
                         2 * A_CHUNK_ATOMS)])
    return 0

  lax.fori_loop(0, n2, pair, 0)


@functools.partial(
    pl.kernel,
    out_type=jax.ShapeDtypeStruct((N_BONDS, HIDDEN), jnp.float32),
    mesh=_sc_mesh,
    scratch_types=[
        pltpu.VMEM((B_CHUNKS, B_CHUNK), jnp.int32),
        pltpu.VMEM((B_CHUNKS, B_CHUNK), jnp.int32),
        pltpu.VMEM((B_CHUNK, HIDDEN), jnp.float32),
        pltpu.VMEM((B_CHUNK, HIDDEN), jnp.float32),
        pltpu.VMEM((B_CHUNK, HIDDEN), jnp.float32),
        pltpu.VMEM((B_CHUNK, HIDDEN), jnp.float32),
        pltpu.SemaphoreType.DMA,
        pltpu.SemaphoreType.DMA,
        pltpu.SemaphoreType.DMA,
        pltpu.SemaphoreType.DMA,
    ],
)
def _sc_edge_msg(a_msg_hbm, msg_hbm, b2a_hbm, brev_hbm, h_hbm,
                 b2a_v, brev_v, am0, rev0, am1, rev1,
                 sem_a0, sem_b0, sem_a1, sem_b1):
  wid = _worker_id()
  base_e = wid * E_PER_W
  pltpu.sync_copy(b2a_hbm.at[wid], b2a_v)
  pltpu.sync_copy(brev_hbm.at[wid], brev_v)

  def issue(ci, am, rev, sa, sb):
    pltpu.async_copy(a_msg_hbm.at[b2a_v.at[ci]], am, sa)
    pltpu.async_copy(msg_hbm.at[brev_v.at[ci]], rev, sb)

  def drain(ci, am, rev, sa, sb):
    pltpu.make_async_copy(a_msg_hbm.at[b2a_v.at[ci]], am, sa).wait()
    pltpu.make_async_copy(msg_hbm.at[brev_v.at[ci]], rev, sb).wait()

  def sub_write(ci, am, rev):
    def sub(t, _):
      e = t // (HIDDEN // LANES) * 4
      col = (t % (HIDDEN // LANES)) * LANES
      for q in range(4):
        am[e + q, pl.ds(col, LANES)] = (
            am[e + q, pl.ds(col, LANES)] - rev[e + q, pl.ds(col, LANES)])
      return 0

    lax.fori_loop(0, B_CHUNK * (HIDDEN // LANES) // 4, sub, 0)
    pltpu.sync_copy(am, h_hbm.at[pl.ds(base_e + ci * B_CHUNK, B_CHUNK)])

  n2 = B_CHUNKS // 2
  issue(0, am0, rev0, sem_a0, sem_b0)

  def pair(c2, _):
    c0 = 2 * c2
    issue(c0 + 1, am1, rev1, sem_a1, sem_b1)
    drain(c0, am0, rev0, sem_a0, sem_b0)
    sub_write(c0, am0, rev0)
    issue(c0 + 2, am0, rev0, sem_a0, sem_b0)
    drain(c0 + 1, am1, rev1, sem_a1, sem_b1)
    sub_write(c0 + 1, am1, rev1)
    return 0

  lax.fori_loop(0, n2, pair, 0)
  last = 2 * n2
  drain(last, am0, rev0, sem_a0, sem_b0)
  sub_write(last, am0, rev0)


E_TILE = 512


def _init_body(fb_ref, wi_ref, wih_ref, bih_ref, inp_ref, gi_ref):
  fb = fb_ref[...]
  inp = lax.dot_general(fb, wi_ref[...], (((1,), (1,)), ((), ())),
                        preferred_element_type=jnp.float32)
  inp_ref[...] = inp
  gi_ref[...] = lax.dot_general(inp, wih_ref[...], (((1,), (1,)), ((), ())),
                                preferred_element_type=jnp.float32) + bih_ref[...]


def _tc_init(f_bonds, w_i, gru_w_ih, gru_b_ih):
  grid = (N_BONDS // E_TILE,)
  return pl.pallas_call(
      _init_body,
      grid=grid,
      in_specs=[
          pl.BlockSpec((E_TILE, BOND_FDIM), lambda i: (i, 0)),
          pl.BlockSpec((HIDDEN, BOND_FDIM), lambda i: (0, 0)),
          pl.BlockSpec((3 * HIDDEN, HIDDEN), lambda i: (0, 0)),
          pl.BlockSpec((1, 3 * HIDDEN), lambda i: (0, 0)),
      ],
      out_specs=[
          pl.BlockSpec((E_TILE, HIDDEN), lambda i: (i, 0)),
          pl.BlockSpec((E_TILE, 3 * HIDDEN), lambda i: (i, 0)),
      ],
      out_shape=[
          jax.ShapeDtypeStruct((N_BONDS, HIDDEN), jnp.float32),
          jax.ShapeDtypeStruct((N_BONDS, 3 * HIDDEN), jnp.float32),
      ],
  )(f_bonds, w_i, gru_w_ih, gru_b_ih.reshape(1, -1))


def _gru_body(gi_ref, h_ref, whh_ref, bhh_ref, out_ref):
  h = h_ref[...]
  gh = lax.dot_general(h, whh_ref[...], (((1,), (1,)), ((), ())),
                       preferred_element_type=jnp.float32) + bhh_ref[...]
  gi = gi_ref[...]
  r = jax.nn.sigmoid(gi[:, :HIDDEN] + gh[:, :HIDDEN])
  z = jax.nn.sigmoid(gi[:, HIDDEN:2 * HIDDEN] + gh[:, HIDDEN:2 * HIDDEN])
  n = jnp.tanh(gi[:, 2 * HIDDEN:] + r * gh[:, 2 * HIDDEN:])
  m = (1.0 - z) * n + z * h
  row = lax.broadcasted_iota(jnp.int32, m.shape, 0)
  is_first = (pl.program_id(0) == 0)
  m = jnp.where(jnp.logical_and(row == 0, is_first), 0.0, m)
  out_ref[...] = m


def _tc_gru(gi, h, gru_w_hh, gru_b_hh):
  grid = (N_BONDS // E_TILE,)
  return pl.pallas_call(
      _gru_body,
      grid=grid,
      in_specs=[
          pl.BlockSpec((E_TILE, 3 * HIDDEN), lambda i: (i, 0)),
          pl.BlockSpec((E_TILE, HIDDEN), lambda i: (i, 0)),
          pl.BlockSpec((3 * HIDDEN, HIDDEN), lambda i: (0, 0)),
          pl.BlockSpec((1, 3 * HIDDEN), lambda i: (0, 0)),
      ],
      out_specs=pl.BlockSpec((E_TILE, HIDDEN), lambda i: (i, 0)),
      out_shape=jax.ShapeDtypeStruct((N_BONDS, HIDDEN), jnp.float32),
  )(gi, h, gru_w_hh, gru_b_hh.reshape(1, -1))


N_TILE = 1000


def _out_body(fa_ref, am_ref, wa_ref, wm_ref, b_ref, out_ref):
  acc = lax.dot_general(fa_ref[...], wa_ref[...], (((1,), (1,)), ((), ())),
                        preferred_element_type=jnp.float32)
  acc = acc + lax.dot_general(am_ref[...], wm_ref[...], (((1,), (1,)), ((), ())),
                              preferred_element_type=jnp.float32)
  out_ref[...] = jnp.maximum(acc + b_ref[...], 0.0)


def _tc_out(f_atoms, a_msg, W_o_w, W_o_b):
  grid = (N_ATOMS // N_TILE,)
  wa = W_o_w[:, :ATOM_FDIM]
  wm = W_o_w[:, ATOM_FDIM:]
  return pl.pallas_call(
      _out_body,
      grid=grid,
      in_specs=[
          pl.BlockSpec((N_TILE, ATOM_FDIM), lambda i: (i, 0)),
          pl.BlockSpec((N_TILE, HIDDEN), lambda i: (i, 0)),
          pl.BlockSpec((HIDDEN, ATOM_FDIM), lambda i: (0, 0)),
          pl.BlockSpec((HIDDEN, HIDDEN), lambda i: (0, 0)),
          pl.BlockSpec((1, HIDDEN), lambda i: (0, 0)),
      ],
      out_specs=pl.BlockSpec((N_TILE, HIDDEN), lambda i: (i, 0)),
      out_shape=jax.ShapeDtypeStruct((N_ATOMS, HIDDEN), jnp.float32),
  )(f_atoms, a_msg, wa, wm, W_o_b.reshape(1, -1))


def kernel(f_atoms, f_bonds, a2b, b2a, b2revb, undirected_b2a,
           w_i, gru_w_ih, gru_w_hh, gru_b_ih, gru_b_hh, W_o_w, W_o_b):
  a2b_pad = jnp.zeros((NPAD, MAX_NB), jnp.int32).at[:N_ATOMS].set(
      a2b.astype(jnp.int32))
  a2b_rs = a2b_pad.reshape(NW, A_CHUNKS, A_IDX_ROW)
  b2a_rs = b2a.astype(jnp.int32).reshape(NW, B_CHUNKS, B_CHUNK)
  brev_rs = b2revb.astype(jnp.int32).reshape(NW, B_CHUNKS, B_CHUNK)

  inp, gi = _tc_init(f_bonds, w_i, gru_w_ih, gru_b_ih)
  msg = inp
  for _ in range(DEPTH - 1):
    a_msg = _sc_gather_sum(msg, a2b_rs)
    h = _sc_edge_msg(a_msg, msg, b2a_rs, brev_rs)
    msg = _tc_gru(gi, h, gru_w_hh, gru_b_hh)
  a_msg = _sc_gather_sum(msg, a2b_rs)
  return _tc_out(f_atoms, a_msg[:N_ATOMS], W_o_w, W_o_b)


# speedup vs baseline: 1.2832x; 1.2832x over previous
import functools

import jax
import jax.numpy as jnp
from jax import lax
from jax.experimental import pallas as pl
from jax.experimental.pallas import tpu as pltpu
from jax.experimental.pallas import tpu_sc as plsc

N_ATOMS = 10000
N_BONDS = 320000
MAX_NB = 32
ATOM_FDIM = 128
BOND_FDIM = 144
HIDDEN = 256
DEPTH = 4

NC = 2
NS = 16
NW = NC * NS
LANES = 16

AT_PER_W = 320
NPAD = NW * AT_PER_W
A_IDX_ROW = 128
A_CHUNK_ATOMS = A_IDX_ROW // MAX_NB
A_CHUNKS = AT_PER_W // A_CHUNK_ATOMS

E_PER_W = N_BONDS // NW
B_CHUNK = 80
B_CHUNKS = E_PER_W // B_CHUNK

_sc_mesh = plsc.VectorSubcoreMesh(
    core_axis_name="c", subcore_axis_name="s", num_cores=NC, num_subcores=NS)


def _worker_id():
  return lax.axis_index("s") * NC + lax.axis_index("c")


@functools.partial(
    pl.kernel,
    out_type=jax.ShapeDtypeStruct((NPAD, HIDDEN), jnp.float32),
    mesh=_sc_mesh,
    scratch_types=[
        pltpu.VMEM((A_CHUNKS, A_IDX_ROW), jnp.int32),
        pltpu.VMEM((A_IDX_ROW, HIDDEN), jnp.float32),
        pltpu.VMEM((A_IDX_ROW, HIDDEN), jnp.float32),
        pltpu.VMEM((2 * A_CHUNK_ATOMS, HIDDEN), jnp.float32),
        pltpu.SemaphoreType.DMA,
        pltpu.SemaphoreType.DMA,
    ],
)
def _sc_gather_sum(msg_hbm, a2b_hbm, out_hbm, idx_v, rows0, rows1, sum_v,
                   sem0, sem1):
  wid = _worker_id()
  base_atom = wid * AT_PER_W
  pltpu.sync_copy(a2b_hbm.at[wid], idx_v)

  def issue(ci, buf, sem):
    pltpu.async_copy(msg_hbm.at[idx_v.at[ci]], buf, sem)

  def drain(ci, buf, sem):
    pltpu.make_async_copy(msg_hbm.at[idx_v.at[ci]], buf, sem).wait()

  def reduce_into(buf, atom_off):
    def red(a, _):
      base = a * MAX_NB
      for j in range(HIDDEN // LANES):
        col = j * LANES
        vals = [buf[base + r, pl.ds(col, LANES)] for r in range(MAX_NB)]
        while len(vals) > 1:
          vals = [vals[i] + vals[i + 1] for i in range(0, len(vals), 2)]
        sum_v[atom_off + a, pl.ds(col, LANES)] = vals[0]
      return 0

    lax.fori_loop(0, A_CHUNK_ATOMS, red, 0)

  n2 = A_CHUNKS // 2
  issue(0, rows0, sem0)
  issue(1, rows1, sem1)

  def pair(c2, _):
    c0 = 2 * c2
    drain(c0, rows0, sem0)
    reduce_into(rows0, 0)

    @pl.when(c0 + 2 < A_CHUNKS)
    def _():
      issue(c0 + 2, rows0, sem0)

    drain(c0 + 1, rows1, sem1)
    reduce_into(rows1, A_CHUNK_ATOMS)

    @pl.when(c0 + 3 < A_CHUNKS)
    def _():
      issue(c0 + 3, rows1, sem1)

    pltpu.sync_copy(
        sum_v,
        out_hbm.at[pl.ds(base_atom + c2 * 2 * A_CHUNK_ATOMS,
                         2 * A_CHUNK_ATOMS)])
    return 0

  lax.fori_loop(0, n2, pair, 0)


@functools.partial(
    pl.kernel,
    out_type=jax.ShapeDtypeStruct((N_BONDS, HIDDEN), jnp.float32),
    mesh=_sc_mesh,
    scratch_types=[
        pltpu.VMEM((B_CHUNKS, B_CHUNK), jnp.int32),
        pltpu.VMEM((B_CHUNKS, B_CHUNK), jnp.int32),
        pltpu.VMEM((B_CHUNK, HIDDEN), jnp.float32),
        pltpu.VMEM((B_CHUNK, HIDDEN), jnp.float32),
        pltpu.VMEM((B_CHUNK, HIDDEN), jnp.float32),
        pltpu.VMEM((B_CHUNK, HIDDEN), jnp.float32),
        pltpu.SemaphoreType.DMA,
        pltpu.SemaphoreType.DMA,
        pltpu.SemaphoreType.DMA,
        pltpu.SemaphoreType.DMA,
    ],
)
def _sc_edge_msg(a_msg_hbm, msg_hbm, b2a_hbm, brev_hbm, h_hbm,
                 b2a_v, brev_v, am0, rev0, am1, rev1,
                 sem_a0, sem_b0, sem_a1, sem_b1):
  wid = _worker_id()
  base_e = wid * E_PER_W
  pltpu.sync_copy(b2a_hbm.at[wid], b2a_v)
  pltpu.sync_copy(brev_hbm.at[wid], brev_v)

  def issue(ci, am, rev, sa, sb):
    pltpu.async_copy(a_msg_hbm.at[b2a_v.at[ci]], am, sa)
    pltpu.async_copy(msg_hbm.at[brev_v.at[ci]], rev, sb)

  def drain(ci, am, rev, sa, sb):
    pltpu.make_async_copy(a_msg_hbm.at[b2a_v.at[ci]], am, sa).wait()
    pltpu.make_async_copy(msg_hbm.at[brev_v.at[ci]], rev, sb).wait()

  def sub_write(ci, am, rev):
    def sub(t, _):
      e = t // (HIDDEN // LANES) * 4
      col = (t % (HIDDEN // LANES)) * LANES
      for q in range(4):
        am[e + q, pl.ds(col, LANES)] = (
            am[e + q, pl.ds(col, LANES)] - rev[e + q, pl.ds(col, LANES)])
      return 0

    lax.fori_loop(0, B_CHUNK * (HIDDEN // LANES) // 4, sub, 0)
    pltpu.sync_copy(am, h_hbm.at[pl.ds(base_e + ci * B_CHUNK, B_CHUNK)])

  n2 = B_CHUNKS // 2
  issue(0, am0, rev0, sem_a0, sem_b0)

  def pair(c2, _):
    c0 = 2 * c2
    issue(c0 + 1, am1, rev1, sem_a1, sem_b1)
    drain(c0, am0, rev0, sem_a0, sem_b0)
    sub_write(c0, am0, rev0)
    issue(c0 + 2, am0, rev0, sem_a0, sem_b0)
    drain(c0 + 1, am1, rev1, sem_a1, sem_b1)
    sub_write(c0 + 1, am1, rev1)
    return 0

  lax.fori_loop(0, n2, pair, 0)
  last = 2 * n2
  drain(last, am0, rev0, sem_a0, sem_b0)
  sub_write(last, am0, rev0)


E_TILE = 512


def _init_body(fb_ref, wi_ref, wih_ref, bih_ref, inp_ref, gi_ref):
  fb = fb_ref[...]
  inp = lax.dot_general(fb, wi_ref[...], (((1,), (1,)), ((), ())),
                        preferred_element_type=jnp.float32)
  inp_ref[...] = inp
  gi_ref[...] = lax.dot_general(inp, wih_ref[...], (((1,), (1,)), ((), ())),
                                preferred_element_type=jnp.float32) + bih_ref[...]


def _tc_init(f_bonds, w_i, gru_w_ih, gru_b_ih):
  grid = (N_BONDS // E_TILE,)
  return pl.pallas_call(
      _init_body,
      grid=grid,
      in_specs=[
          pl.BlockSpec((E_TILE, BOND_FDIM), lambda i: (i, 0)),
          pl.BlockSpec((HIDDEN, BOND_FDIM), lambda i: (0, 0)),
          pl.BlockSpec((3 * HIDDEN, HIDDEN), lambda i: (0, 0)),
          pl.BlockSpec((1, 3 * HIDDEN), lambda i: (0, 0)),
      ],
      out_specs=[
          pl.BlockSpec((E_TILE, HIDDEN), lambda i: (i, 0)),
          pl.BlockSpec((E_TILE, 3 * HIDDEN), lambda i: (i, 0)),
      ],
      out_shape=[
          jax.ShapeDtypeStruct((N_BONDS, HIDDEN), jnp.float32),
          jax.ShapeDtypeStruct((N_BONDS, 3 * HIDDEN), jnp.float32),
      ],
  )(f_bonds, w_i, gru_w_ih, gru_b_ih.reshape(1, -1))


def _gru_body(gi_ref, h_ref, whh_ref, bhh_ref, out_ref):
  h = h_ref[...]
  gh = lax.dot_general(h, whh_ref[...], (((1,), (1,)), ((), ())),
                       preferred_element_type=jnp.float32) + bhh_ref[...]
  gi = gi_ref[...]
  r = jax.nn.sigmoid(gi[:, :HIDDEN] + gh[:, :HIDDEN])
  z = jax.nn.sigmoid(gi[:, HIDDEN:2 * HIDDEN] + gh[:, HIDDEN:2 * HIDDEN])
  n = jnp.tanh(gi[:, 2 * HIDDEN:] + r * gh[:, 2 * HIDDEN:])
  m = (1.0 - z) * n + z * h
  row = lax.broadcasted_iota(jnp.int32, m.shape, 0)
  is_first = (pl.program_id(0) == 0)
  m = jnp.where(jnp.logical_and(row == 0, is_first), 0.0, m)
  out_ref[...] = m


def _tc_gru(gi, h, gru_w_hh, gru_b_hh):
  grid = (N_BONDS // E_TILE,)
  return pl.pallas_call(
      _gru_body,
      grid=grid,
      in_specs=[
          pl.BlockSpec((E_TILE, 3 * HIDDEN), lambda i: (i, 0)),
          pl.BlockSpec((E_TILE, HIDDEN), lambda i: (i, 0)),
          pl.BlockSpec((3 * HIDDEN, HIDDEN), lambda i: (0, 0)),
          pl.BlockSpec((1, 3 * HIDDEN), lambda i: (0, 0)),
      ],
      out_specs=pl.BlockSpec((E_TILE, HIDDEN), lambda i: (i, 0)),
      out_shape=jax.ShapeDtypeStruct((N_BONDS, HIDDEN), jnp.float32),
  )(gi, h, gru_w_hh, gru_b_hh.reshape(1, -1))


N_TILE = 1000


def _out_body(fa_ref, am_ref, wa_ref, wm_ref, b_ref, out_ref):
  acc = lax.dot_general(fa_ref[...], wa_ref[...], (((1,), (1,)), ((), ())),
                        preferred_element_type=jnp.float32)
  acc = acc + lax.dot_general(am_ref[...], wm_ref[...], (((1,), (1,)), ((), ())),
                              preferred_element_type=jnp.float32)
  out_ref[...] = jnp.maximum(acc + b_ref[...], 0.0)


def _tc_out(f_atoms, a_msg, W_o_w, W_o_b):
  grid = (N_ATOMS // N_TILE,)
  wa = W_o_w[:, :ATOM_FDIM]
  wm = W_o_w[:, ATOM_FDIM:]
  return pl.pallas_call(
      _out_body,
      grid=grid,
      in_specs=[
          pl.BlockSpec((N_TILE, ATOM_FDIM), lambda i: (i, 0)),
          pl.BlockSpec((N_TILE, HIDDEN), lambda i: (i, 0)),
          pl.BlockSpec((HIDDEN, ATOM_FDIM), lambda i: (0, 0)),
          pl.BlockSpec((HIDDEN, HIDDEN), lambda i: (0, 0)),
          pl.BlockSpec((1, HIDDEN), lambda i: (0, 0)),
      ],
      out_specs=pl.BlockSpec((N_TILE, HIDDEN), lambda i: (i, 0)),
      out_shape=jax.ShapeDtypeStruct((N_ATOMS, HIDDEN), jnp.float32),
  )(f_atoms, a_msg, wa, wm, W_o_b.reshape(1, -1))


def kernel(f_atoms, f_bonds, a2b, b2a, b2revb, undirected_b2a,
           w_i, gru_w_ih, gru_w_hh, gru_b_ih, gru_b_hh, W_o_w, W_o_b):
  pad_idx = jnp.arange(
      (NPAD - N_ATOMS) * MAX_NB, dtype=jnp.int32).reshape(
          NPAD - N_ATOMS, MAX_NB)
  a2b_pad = jnp.concatenate([a2b.astype(jnp.int32), pad_idx], axis=0)
  a2b_rs = a2b_pad.reshape(NW, A_CHUNKS, A_IDX_ROW)
  b2a_rs = b2a.astype(jnp.int32).reshape(NW, B_CHUNKS, B_CHUNK)
  brev_rs = b2revb.astype(jnp.int32).reshape(NW, B_CHUNKS, B_CHUNK)

  inp, gi = _tc_init(f_bonds, w_i, gru_w_ih, gru_b_ih)
  msg = inp
  for _ in range(DEPTH - 1):
    a_msg = _sc_gather_sum(msg, a2b_rs)
    h = _sc_edge_msg(a_msg, msg, b2a_rs, brev_rs)
    msg = _tc_gru(gi, h, gru_w_hh, gru_b_hh)
  a_msg = _sc_gather_sum(msg, a2b_rs)
  return _tc_out(f_atoms, a_msg[:N_ATOMS], W_o_w, W_o_b)


# speedup vs baseline: 1.3485x; 1.0509x over previous
import functools

import jax
import jax.numpy as jnp
from jax import lax
from jax.experimental import pallas as pl
from jax.experimental.pallas import tpu as pltpu
from jax.experimental.pallas import tpu_sc as plsc

N_ATOMS = 10000
N_BONDS = 320000
MAX_NB = 32
ATOM_FDIM = 128
BOND_FDIM = 144
HIDDEN = 256
DEPTH = 4

NC = 2
NS = 16
NW = NC * NS
LANES = 16
PLANES = 2 * LANES

AT_PER_W = 320
NPAD = NW * AT_PER_W
A_IDX_ROW = 128
A_CHUNK_ATOMS = A_IDX_ROW // MAX_NB
A_CHUNKS = AT_PER_W // A_CHUNK_ATOMS

E_PER_W = N_BONDS // NW
B_CHUNK = 80
B_CHUNKS = E_PER_W // B_CHUNK

_sc_mesh = plsc.VectorSubcoreMesh(
    core_axis_name="c", subcore_axis_name="s", num_cores=NC, num_subcores=NS)


def _worker_id():
  return lax.axis_index("s") * NC + lax.axis_index("c")


@functools.partial(
    pl.kernel,
    out_type=jax.ShapeDtypeStruct((NPAD, HIDDEN), jnp.float32),
    mesh=_sc_mesh,
    scratch_types=[
        pltpu.VMEM((A_CHUNKS, A_IDX_ROW), jnp.int32),
        pltpu.VMEM((A_IDX_ROW, HIDDEN), jnp.float32),
        pltpu.VMEM((A_IDX_ROW, HIDDEN), jnp.float32),
        pltpu.VMEM((4 * A_CHUNK_ATOMS, HIDDEN), jnp.float32),
        pltpu.SemaphoreType.DMA,
        pltpu.SemaphoreType.DMA,
    ],
)
def _sc_gather_sum(msg_hbm, a2b_hbm, out_hbm, idx_v, rows0, rows1, sum_v,
                   sem0, sem1):
  wid = _worker_id()
  base_atom = wid * AT_PER_W
  pltpu.sync_copy(a2b_hbm.at[wid], idx_v)

  def issue(ci, buf, sem):
    pltpu.async_copy(msg_hbm.at[idx_v.at[ci]], buf, sem)

  def drain(ci, buf, sem):
    pltpu.make_async_copy(msg_hbm.at[idx_v.at[ci]], buf, sem).wait()

  def reduce_into(buf, atom_off):
    def red(a, _):
      base = a * MAX_NB
      for j in range(HIDDEN // LANES):
        col = j * LANES
        vals = [buf[base + r, pl.ds(col, LANES)] for r in range(MAX_NB)]
        while len(vals) > 1:
          vals = [vals[i] + vals[i + 1] for i in range(0, len(vals), 2)]
        sum_v[atom_off + a, pl.ds(col, LANES)] = vals[0]
      return 0

    lax.fori_loop(0, A_CHUNK_ATOMS, red, 0)

  nq = A_CHUNKS // 4
  issue(0, rows0, sem0)
  issue(1, rows1, sem1)

  def quad(g, _):
    c0 = 4 * g
    drain(c0, rows0, sem0)
    reduce_into(rows0, 0)
    issue(c0 + 2, rows0, sem0)
    drain(c0 + 1, rows1, sem1)
    reduce_into(rows1, A_CHUNK_ATOMS)
    issue(c0 + 3, rows1, sem1)
    drain(c0 + 2, rows0, sem0)
    reduce_into(rows0, 2 * A_CHUNK_ATOMS)

    @pl.when(c0 + 4 < A_CHUNKS)
    def _():
      issue(c0 + 4, rows0, sem0)

    drain(c0 + 3, rows1, sem1)
    reduce_into(rows1, 3 * A_CHUNK_ATOMS)

    @pl.when(c0 + 5 < A_CHUNKS)
    def _():
      issue(c0 + 5, rows1, sem1)

    pltpu.sync_copy(
        sum_v,
        out_hbm.at[pl.ds(base_atom + g * 4 * A_CHUNK_ATOMS,
                         4 * A_CHUNK_ATOMS)])
    return 0

  lax.fori_loop(0, nq, quad, 0)


@functools.partial(
    pl.kernel,
    out_type=jax.ShapeDtypeStruct((N_BONDS, HIDDEN), jnp.float32),
    mesh=_sc_mesh,
    scratch_types=[
        pltpu.VMEM((B_CHUNKS, B_CHUNK), jnp.int32),
        pltpu.VMEM((B_CHUNKS, B_CHUNK), jnp.int32),
        pltpu.VMEM((B_CHUNK, HIDDEN), jnp.float32),
        pltpu.VMEM((B_CHUNK, HIDDEN), jnp.float32),
        pltpu.VMEM((B_CHUNK, HIDDEN), jnp.float32),
        pltpu.VMEM((B_CHUNK, HIDDEN), jnp.float32),
        pltpu.SemaphoreType.DMA,
        pltpu.SemaphoreType.DMA,
        pltpu.SemaphoreType.DMA,
        pltpu.SemaphoreType.DMA,
    ],
)
def _sc_edge_msg(a_msg_hbm, msg_hbm, b2a_hbm, brev_hbm, h_hbm,
                 b2a_v, brev_v, am0, rev0, am1, rev1,
                 sem_a0, sem_b0, sem_a1, sem_b1):
  wid = _worker_id()
  base_e = wid * E_PER_W
  pltpu.sync_copy(b2a_hbm.at[wid], b2a_v)
  pltpu.sync_copy(brev_hbm.at[wid], brev_v)

  def issue(ci, am, rev, sa, sb):
    pltpu.async_copy(a_msg_hbm.at[b2a_v.at[ci]], am, sa)
    pltpu.async_copy(msg_hbm.at[brev_v.at[ci]], rev, sb)

  def drain(ci, am, rev, sa, sb):
    pltpu.make_async_copy(a_msg_hbm.at[b2a_v.at[ci]], am, sa).wait()
    pltpu.make_async_copy(msg_hbm.at[brev_v.at[ci]], rev, sb).wait()

  def sub_write(ci, am, rev):
    def sub(t, _):
      e = t // (HIDDEN // LANES) * 4
      col = (t % (HIDDEN // LANES)) * LANES
      for q in range(4):
        am[e + q, pl.ds(col, LANES)] = (
            am[e + q, pl.ds(col, LANES)] - rev[e + q, pl.ds(col, LANES)])
      return 0

    lax.fori_loop(0, B_CHUNK * (HIDDEN // LANES) // 4, sub, 0)
    pltpu.sync_copy(am, h_hbm.at[pl.ds(base_e + ci * B_CHUNK, B_CHUNK)])

  n2 = B_CHUNKS // 2
  issue(0, am0, rev0, sem_a0, sem_b0)

  def pair(c2, _):
    c0 = 2 * c2
    issue(c0 + 1, am1, rev1, sem_a1, sem_b1)
    drain(c0, am0, rev0, sem_a0, sem_b0)
    sub_write(c0, am0, rev0)
    issue(c0 + 2, am0, rev0, sem_a0, sem_b0)
    drain(c0 + 1, am1, rev1, sem_a1, sem_b1)
    sub_write(c0 + 1, am1, rev1)
    return 0

  lax.fori_loop(0, n2, pair, 0)
  last = 2 * n2
  drain(last, am0, rev0, sem_a0, sem_b0)
  sub_write(last, am0, rev0)


E_TILE = 512


def _init_body(fb_ref, wi_ref, wih_ref, bih_ref, inp_ref, gi_ref):
  fb = fb_ref[...]
  inp = lax.dot_general(fb, wi_ref[...], (((1,), (1,)), ((), ())),
                        preferred_element_type=jnp.float32)
  inp_ref[...] = inp
  gi = lax.dot_general(inp, wih_ref[...], (((1,), (1,)), ((), ())),
                       preferred_element_type=jnp.float32) + bih_ref[...]
  gi_ref[...] = gi.astype(jnp.bfloat16)


def _tc_init(f_bonds, w_i, gru_w_ih, gru_b_ih):
  grid = (N_BONDS // E_TILE,)
  return pl.pallas_call(
      _init_body,
      grid=grid,
      in_specs=[
          pl.BlockSpec((E_TILE, BOND_FDIM), lambda i: (i, 0)),
          pl.BlockSpec((HIDDEN, BOND_FDIM), lambda i: (0, 0)),
          pl.BlockSpec((3 * HIDDEN, HIDDEN), lambda i: (0, 0)),
          pl.BlockSpec((1, 3 * HIDDEN), lambda i: (0, 0)),
      ],
      out_specs=[
          pl.BlockSpec((E_TILE, HIDDEN), lambda i: (i, 0)),
          pl.BlockSpec((E_TILE, 3 * HIDDEN), lambda i: (i, 0)),
      ],
      out_shape=[
          jax.ShapeDtypeStruct((N_BONDS, HIDDEN), jnp.float32),
          jax.ShapeDtypeStruct((N_BONDS, 3 * HIDDEN), jnp.bfloat16),
      ],
  )(f_bonds, w_i, gru_w_ih, gru_b_ih.reshape(1, -1))


def _gru_body(gi_ref, h_ref, whh_ref, bhh_ref, out_ref):
  h = h_ref[...]
  gh = lax.dot_general(h, whh_ref[...], (((1,), (1,)), ((), ())),
                       preferred_element_type=jnp.float32) + bhh_ref[...]
  gi = gi_ref[...].astype(jnp.float32)
  r = jax.nn.sigmoid(gi[:, :HIDDEN] + gh[:, :HIDDEN])
  z = jax.nn.sigmoid(gi[:, HIDDEN:2 * HIDDEN] + gh[:, HIDDEN:2 * HIDDEN])
  n = jnp.tanh(gi[:, 2 * HIDDEN:] + r * gh[:, 2 * HIDDEN:])
  m = (1.0 - z) * n + z * h
  row = lax.broadcasted_iota(jnp.int32, m.shape, 0)
  is_first = (pl.program_id(0) == 0)
  m = jnp.where(jnp.logical_and(row == 0, is_first), 0.0, m)
  out_ref[...] = m


def _tc_gru(gi, h, gru_w_hh, gru_b_hh):
  grid = (N_BONDS // E_TILE,)
  return pl.pallas_call(
      _gru_body,
      grid=grid,
      in_specs=[
          pl.BlockSpec((E_TILE, 3 * HIDDEN), lambda i: (i, 0)),
          pl.BlockSpec((E_TILE, HIDDEN), lambda i: (i, 0)),
          pl.BlockSpec((3 * HIDDEN, HIDDEN), lambda i: (0, 0)),
          pl.BlockSpec((1, 3 * HIDDEN), lambda i: (0, 0)),
      ],
      out_specs=pl.BlockSpec((E_TILE, HIDDEN), lambda i: (i, 0)),
      out_shape=jax.ShapeDtypeStruct((N_BONDS, HIDDEN), jnp.float32),
  )(gi, h, gru_w_hh, gru_b_hh.reshape(1, -1))


N_TILE = 1000


def _out_body(fa_ref, am_ref, wa_ref, wm_ref, b_ref, out_ref):
  acc = lax.dot_general(fa_ref[...], wa_ref[...], (((1,), (1,)), ((), ())),
                        preferred_element_type=jnp.float32)
  acc = acc + lax.dot_general(am_ref[...], wm_ref[...], (((1,), (1,)), ((), ())),
                              preferred_element_type=jnp.float32)
  out_ref[...] = jnp.maximum(acc + b_ref[...], 0.0)


def _tc_out(f_atoms, a_msg, W_o_w, W_o_b):
  grid = (N_ATOMS // N_TILE,)
  wa = W_o_w[:, :ATOM_FDIM]
  wm = W_o_w[:, ATOM_FDIM:]
  return pl.pallas_call(
      _out_body,
      grid=grid,
      in_specs=[
          pl.BlockSpec((N_TILE, ATOM_FDIM), lambda i: (i, 0)),
          pl.BlockSpec((N_TILE, HIDDEN), lambda i: (i, 0)),
          pl.BlockSpec((HIDDEN, ATOM_FDIM), lambda i: (0, 0)),
          pl.BlockSpec((HIDDEN, HIDDEN), lambda i: (0, 0)),
          pl.BlockSpec((1, HIDDEN), lambda i: (0, 0)),
      ],
      out_specs=pl.BlockSpec((N_TILE, HIDDEN), lambda i: (i, 0)),
      out_shape=jax.ShapeDtypeStruct((N_ATOMS, HIDDEN), jnp.float32),
  )(f_atoms, a_msg, wa, wm, W_o_b.reshape(1, -1))


def kernel(f_atoms, f_bonds, a2b, b2a, b2revb, undirected_b2a,
           w_i, gru_w_ih, gru_w_hh, gru_b_ih, gru_b_hh, W_o_w, W_o_b):
  pad_idx = jnp.arange(
      (NPAD - N_ATOMS) * MAX_NB, dtype=jnp.int32).reshape(
          NPAD - N_ATOMS, MAX_NB)
  a2b_pad = jnp.concatenate([a2b.astype(jnp.int32), pad_idx], axis=0)
  a2b_rs = a2b_pad.reshape(NW, A_CHUNKS, A_IDX_ROW)
  b2a_rs = b2a.astype(jnp.int32).reshape(NW, B_CHUNKS, B_CHUNK)
  brev_rs = b2revb.astype(jnp.int32).reshape(NW, B_CHUNKS, B_CHUNK)

  inp, gi = _tc_init(f_bonds, w_i, gru_w_ih, gru_b_ih)
  msg = inp
  for _ in range(DEPTH - 1):
    a_msg = _sc_gather_sum(msg, a2b_rs)
    h = _sc_edge_msg(a_msg, msg, b2a_rs, brev_rs)
    msg = _tc_gru(gi, h, gru_w_hh, gru_b_hh)
  a_msg = _sc_gather_sum(msg, a2b_rs)
  return _tc_out(f_atoms, a_msg[:N_ATOMS], W_o_w, W_o_b)


# speedup vs baseline: 1.6108x; 1.1945x over previous
import functools

import jax
import jax.numpy as jnp
from jax import lax
from jax.experimental import pallas as pl
from jax.experimental.pallas import tpu as pltpu
from jax.experimental.pallas import tpu_sc as plsc

N_ATOMS = 10000
N_BONDS = 320000
MAX_NB = 32
ATOM_FDIM = 128
BOND_FDIM = 144
HIDDEN = 256
DEPTH = 4

NC = 2
NS = 16
NW = NC * NS
LANES = 16
PLANES = 2 * LANES

AT_PER_W = 320
NPAD = NW * AT_PER_W
A_IDX_ROW = 128
A_CHUNK_ATOMS = A_IDX_ROW // MAX_NB
A_CHUNKS = AT_PER_W // A_CHUNK_ATOMS

E_PER_W = N_BONDS // NW
B_CHUNK = 80
B_CHUNKS = E_PER_W // B_CHUNK

_sc_mesh = plsc.VectorSubcoreMesh(
    core_axis_name="c", subcore_axis_name="s", num_cores=NC, num_subcores=NS)


def _worker_id():
  return lax.axis_index("s") * NC + lax.axis_index("c")


@functools.partial(
    pl.kernel,
    out_type=jax.ShapeDtypeStruct((NPAD, HIDDEN), jnp.float32),
    mesh=_sc_mesh,
    scratch_types=[
        pltpu.VMEM((A_CHUNKS, A_IDX_ROW), jnp.int32),
        pltpu.VMEM((A_IDX_ROW, HIDDEN), jnp.float32),
        pltpu.VMEM((A_IDX_ROW, HIDDEN), jnp.float32),
        pltpu.VMEM((4 * A_CHUNK_ATOMS, HIDDEN), jnp.float32),
        pltpu.SemaphoreType.DMA,
        pltpu.SemaphoreType.DMA,
    ],
)
def _sc_gather_sum(msg_hbm, a2b_hbm, out_hbm, idx_v, rows0, rows1, sum_v,
                   sem0, sem1):
  wid = _worker_id()
  base_atom = wid * AT_PER_W
  pltpu.sync_copy(a2b_hbm.at[wid], idx_v)

  def issue(ci, buf, sem):
    pltpu.async_copy(msg_hbm.at[idx_v.at[ci]], buf, sem)

  def drain(ci, buf, sem):
    pltpu.make_async_copy(msg_hbm.at[idx_v.at[ci]], buf, sem).wait()

  def reduce_into(buf, atom_off):
    def red(a, _):
      base = a * MAX_NB
      for j in range(HIDDEN // LANES):
        col = j * LANES
        vals = [buf[base + r, pl.ds(col, LANES)] for r in range(MAX_NB)]
        while len(vals) > 1:
          vals = [vals[i] + vals[i + 1] for i in range(0, len(vals), 2)]
        sum_v[atom_off + a, pl.ds(col, LANES)] = vals[0]
      return 0

    lax.fori_loop(0, A_CHUNK_ATOMS, red, 0)

  nq = A_CHUNKS // 4
  issue(0, rows0, sem0)
  issue(1, rows1, sem1)

  def quad(g, _):
    c0 = 4 * g
    drain(c0, rows0, sem0)
    reduce_into(rows0, 0)
    issue(c0 + 2, rows0, sem0)
    drain(c0 + 1, rows1, sem1)
    reduce_into(rows1, A_CHUNK_ATOMS)
    issue(c0 + 3, rows1, sem1)
    drain(c0 + 2, rows0, sem0)
    reduce_into(rows0, 2 * A_CHUNK_ATOMS)

    @pl.when(c0 + 4 < A_CHUNKS)
    def _():
      issue(c0 + 4, rows0, sem0)

    drain(c0 + 3, rows1, sem1)
    reduce_into(rows1, 3 * A_CHUNK_ATOMS)

    @pl.when(c0 + 5 < A_CHUNKS)
    def _():
      issue(c0 + 5, rows1, sem1)

    pltpu.sync_copy(
        sum_v,
        out_hbm.at[pl.ds(base_atom + g * 4 * A_CHUNK_ATOMS,
                         4 * A_CHUNK_ATOMS)])
    return 0

  lax.fori_loop(0, nq, quad, 0)


@functools.partial(
    pl.kernel,
    out_type=jax.ShapeDtypeStruct((N_BONDS, HIDDEN), jnp.float32),
    mesh=_sc_mesh,
    scratch_types=[
        pltpu.VMEM((B_CHUNKS, B_CHUNK), jnp.int32),
        pltpu.VMEM((B_CHUNKS, B_CHUNK), jnp.int32),
        pltpu.VMEM((B_CHUNK, HIDDEN), jnp.float32),
        pltpu.VMEM((B_CHUNK, HIDDEN), jnp.float32),
        pltpu.VMEM((B_CHUNK, HIDDEN), jnp.float32),
        pltpu.VMEM((B_CHUNK, HIDDEN), jnp.float32),
        pltpu.SemaphoreType.DMA,
        pltpu.SemaphoreType.DMA,
        pltpu.SemaphoreType.DMA,
        pltpu.SemaphoreType.DMA,
    ],
)
def _sc_edge_msg(a_msg_hbm, msg_hbm, b2a_hbm, brev_hbm, h_hbm,
                 b2a_v, brev_v, am0, rev0, am1, rev1,
                 sem_a0, sem_b0, sem_a1, sem_b1):
  wid = _worker_id()
  base_e = wid * E_PER_W
  pltpu.sync_copy(b2a_hbm.at[wid], b2a_v)
  pltpu.sync_copy(brev_hbm.at[wid], brev_v)

  def issue(ci, am, rev, sa, sb):
    pltpu.async_copy(a_msg_hbm.at[b2a_v.at[ci]], am, sa)
    pltpu.async_copy(msg_hbm.at[brev_v.at[ci]], rev, sb)

  def drain(ci, am, rev, sa, sb):
    pltpu.make_async_copy(a_msg_hbm.at[b2a_v.at[ci]], am, sa).wait()
    pltpu.make_async_copy(msg_hbm.at[brev_v.at[ci]], rev, sb).wait()

  def sub_write(ci, am, rev):
    def sub(t, _):
      e = t // (HIDDEN // LANES) * 4
      col = (t % (HIDDEN // LANES)) * LANES
      for q in range(4):
        am[e + q, pl.ds(col, LANES)] = (
            am[e + q, pl.ds(col, LANES)] - rev[e + q, pl.ds(col, LANES)])
      return 0

    lax.fori_loop(0, B_CHUNK * (HIDDEN // LANES) // 4, sub, 0)
    pltpu.sync_copy(am, h_hbm.at[pl.ds(base_e + ci * B_CHUNK, B_CHUNK)])

  n2 = B_CHUNKS // 2
  issue(0, am0, rev0, sem_a0, sem_b0)

  def pair(c2, _):
    c0 = 2 * c2
    issue(c0 + 1, am1, rev1, sem_a1, sem_b1)
    drain(c0, am0, rev0, sem_a0, sem_b0)
    sub_write(c0, am0, rev0)
    issue(c0 + 2, am0, rev0, sem_a0, sem_b0)
    drain(c0 + 1, am1, rev1, sem_a1, sem_b1)
    sub_write(c0 + 1, am1, rev1)
    return 0

  lax.fori_loop(0, n2, pair, 0)
  last = 2 * n2
  drain(last, am0, rev0, sem_a0, sem_b0)
  sub_write(last, am0, rev0)


E_TILE = 1024


def _init_body(fb_ref, wi_ref, wih_ref, bih_ref, inp_ref, gi_ref):
  fb = fb_ref[...]
  inp = lax.dot_general(fb, wi_ref[...], (((1,), (1,)), ((), ())),
                        preferred_element_type=jnp.float32)
  inp_ref[...] = inp
  gi = lax.dot_general(inp, wih_ref[...], (((1,), (1,)), ((), ())),
                       preferred_element_type=jnp.float32) + bih_ref[...]
  gi_ref[...] = gi.astype(jnp.bfloat16)


def _tc_init(f_bonds, w_i, gru_w_ih, gru_b_ih):
  grid = (N_BONDS // E_TILE,)
  return pl.pallas_call(
      _init_body,
      grid=grid,
      in_specs=[
          pl.BlockSpec((E_TILE, BOND_FDIM), lambda i: (i, 0)),
          pl.BlockSpec((HIDDEN, BOND_FDIM), lambda i: (0, 0)),
          pl.BlockSpec((3 * HIDDEN, HIDDEN), lambda i: (0, 0)),
          pl.BlockSpec((1, 3 * HIDDEN), lambda i: (0, 0)),
      ],
      out_specs=[
          pl.BlockSpec((E_TILE, HIDDEN), lambda i: (i, 0)),
          pl.BlockSpec((E_TILE, 3 * HIDDEN), lambda i: (i, 0)),
      ],
      out_shape=[
          jax.ShapeDtypeStruct((N_BONDS, HIDDEN), jnp.float32),
          jax.ShapeDtypeStruct((N_BONDS, 3 * HIDDEN), jnp.bfloat16),
      ],
  )(f_bonds, w_i, gru_w_ih, gru_b_ih.reshape(1, -1))


def _gru_body(gi_ref, h_ref, whh_ref, bhh_ref, out_ref):
  h = h_ref[...]
  gh = lax.dot_general(h, whh_ref[...], (((1,), (1,)), ((), ())),
                       preferred_element_type=jnp.float32) + bhh_ref[...]
  gi = gi_ref[...].astype(jnp.float32)
  r = jax.nn.sigmoid(gi[:, :HIDDEN] + gh[:, :HIDDEN])
  z = jax.nn.sigmoid(gi[:, HIDDEN:2 * HIDDEN] + gh[:, HIDDEN:2 * HIDDEN])
  n = jnp.tanh(gi[:, 2 * HIDDEN:] + r * gh[:, 2 * HIDDEN:])
  m = (1.0 - z) * n + z * h
  row = lax.broadcasted_iota(jnp.int32, m.shape, 0)
  is_first = (pl.program_id(0) == 0)
  m = jnp.where(jnp.logical_and(row == 0, is_first), 0.0, m)
  out_ref[...] = m


def _tc_gru(gi, h, gru_w_hh, gru_b_hh):
  grid = (N_BONDS // E_TILE,)
  return pl.pallas_call(
      _gru_body,
      grid=grid,
      in_specs=[
          pl.BlockSpec((E_TILE, 3 * HIDDEN), lambda i: (i, 0)),
          pl.BlockSpec((E_TILE, HIDDEN), lambda i: (i, 0)),
          pl.BlockSpec((3 * HIDDEN, HIDDEN), lambda i: (0, 0)),
          pl.BlockSpec((1, 3 * HIDDEN), lambda i: (0, 0)),
      ],
      out_specs=pl.BlockSpec((E_TILE, HIDDEN), lambda i: (i, 0)),
      out_shape=jax.ShapeDtypeStruct((N_BONDS, HIDDEN), jnp.float32),
  )(gi, h, gru_w_hh, gru_b_hh.reshape(1, -1))


N_TILE = 1000


def _out_body(fa_ref, am_ref, wa_ref, wm_ref, b_ref, out_ref):
  acc = lax.dot_general(fa_ref[...], wa_ref[...], (((1,), (1,)), ((), ())),
                        preferred_element_type=jnp.float32)
  acc = acc + lax.dot_general(am_ref[...], wm_ref[...], (((1,), (1,)), ((), ())),
                              preferred_element_type=jnp.float32)
  out_ref[...] = jnp.maximum(acc + b_ref[...], 0.0)


def _tc_out(f_atoms, a_msg, W_o_w, W_o_b):
  grid = (N_ATOMS // N_TILE,)
  wa = W_o_w[:, :ATOM_FDIM]
  wm = W_o_w[:, ATOM_FDIM:]
  return pl.pallas_call(
      _out_body,
      grid=grid,
      in_specs=[
          pl.BlockSpec((N_TILE, ATOM_FDIM), lambda i: (i, 0)),
          pl.BlockSpec((N_TILE, HIDDEN), lambda i: (i, 0)),
          pl.BlockSpec((HIDDEN, ATOM_FDIM), lambda i: (0, 0)),
          pl.BlockSpec((HIDDEN, HIDDEN), lambda i: (0, 0)),
          pl.BlockSpec((1, HIDDEN), lambda i: (0, 0)),
      ],
      out_specs=pl.BlockSpec((N_TILE, HIDDEN), lambda i: (i, 0)),
      out_shape=jax.ShapeDtypeStruct((N_ATOMS, HIDDEN), jnp.float32),
  )(f_atoms, a_msg, wa, wm, W_o_b.reshape(1, -1))


def kernel(f_atoms, f_bonds, a2b, b2a, b2revb, undirected_b2a,
           w_i, gru_w_ih, gru_w_hh, gru_b_ih, gru_b_hh, W_o_w, W_o_b):
  pad_idx = jnp.arange(
      (NPAD - N_ATOMS) * MAX_NB, dtype=jnp.int32).reshape(
          NPAD - N_ATOMS, MAX_NB)
  a2b_pad = jnp.concatenate([a2b.astype(jnp.int32), pad_idx], axis=0)
  a2b_rs = a2b_pad.reshape(NW, A_CHUNKS, A_IDX_ROW)
  b2a_rs = b2a.astype(jnp.int32).reshape(NW, B_CHUNKS, B_CHUNK)
  brev_rs = b2revb.astype(jnp.int32).reshape(NW, B_CHUNKS, B_CHUNK)

  inp, gi = _tc_init(f_bonds, w_i, gru_w_ih, gru_b_ih)
  msg = inp
  for _ in range(DEPTH - 1):
    a_msg = _sc_gather_sum(msg, a2b_rs)
    h = _sc_edge_msg(a_msg, msg, b2a_rs, brev_rs)
    msg = _tc_gru(gi, h, gru_w_hh, gru_b_hh)
  a_msg = _sc_gather_sum(msg, a2b_rs)
  return _tc_out(f_atoms, a_msg[:N_ATOMS], W_o_w, W_o_b)


# speedup vs baseline: 1.6824x; 1.0444x over previous
import functools

import jax
import jax.numpy as jnp
from jax import lax
from jax.experimental import pallas as pl
from jax.experimental.pallas import tpu as pltpu
from jax.experimental.pallas import tpu_sc as plsc

N_ATOMS = 10000
N_BONDS = 320000
MAX_NB = 32
ATOM_FDIM = 128
BOND_FDIM = 144
HIDDEN = 256
DEPTH = 4

NC = 2
NS = 16
NW = NC * NS
LANES = 16
PLANES = 2 * LANES

AT_PER_W = 320
NPAD = NW * AT_PER_W
A_IDX_ROW = 128
A_CHUNK_ATOMS = A_IDX_ROW // MAX_NB
A_CHUNKS = AT_PER_W // A_CHUNK_ATOMS

E_PER_W = N_BONDS // NW
B_CHUNK = 80
B_CHUNKS = E_PER_W // B_CHUNK

_sc_mesh = plsc.VectorSubcoreMesh(
    core_axis_name="c", subcore_axis_name="s", num_cores=NC, num_subcores=NS)


def _worker_id():
  return lax.axis_index("s") * NC + lax.axis_index("c")


@functools.partial(
    pl.kernel,
    out_type=jax.ShapeDtypeStruct((NPAD, HIDDEN), jnp.float32),
    mesh=_sc_mesh,
    scratch_types=[
        pltpu.VMEM((A_CHUNKS, A_IDX_ROW), jnp.int32),
        pltpu.VMEM((A_IDX_ROW, HIDDEN), jnp.float32),
        pltpu.VMEM((A_IDX_ROW, HIDDEN), jnp.float32),
        pltpu.VMEM((4 * A_CHUNK_ATOMS, HIDDEN), jnp.float32),
        pltpu.SemaphoreType.DMA,
        pltpu.SemaphoreType.DMA,
    ],
)
def _sc_gather_sum(msg_hbm, a2b_hbm, out_hbm, idx_v, rows0, rows1, sum_v,
                   sem0, sem1):
  wid = _worker_id()
  base_atom = wid * AT_PER_W
  pltpu.sync_copy(a2b_hbm.at[wid], idx_v)

  def issue(ci, buf, sem):
    pltpu.async_copy(msg_hbm.at[idx_v.at[ci]], buf, sem)

  def drain(ci, buf, sem):
    pltpu.make_async_copy(msg_hbm.at[idx_v.at[ci]], buf, sem).wait()

  def reduce_into(buf, atom_off):
    def red(a, _):
      base = a * MAX_NB
      for j in range(HIDDEN // LANES):
        col = j * LANES
        vals = [buf[base + r, pl.ds(col, LANES)] for r in range(MAX_NB)]
        while len(vals) > 1:
          vals = [vals[i] + vals[i + 1] for i in range(0, len(vals), 2)]
        sum_v[atom_off + a, pl.ds(col, LANES)] = vals[0]
      return 0

    lax.fori_loop(0, A_CHUNK_ATOMS, red, 0)

  nq = A_CHUNKS // 4
  issue(0, rows0, sem0)
  issue(1, rows1, sem1)

  def quad(g, _):
    c0 = 4 * g
    drain(c0, rows0, sem0)
    reduce_into(rows0, 0)
    issue(c0 + 2, rows0, sem0)
    drain(c0 + 1, rows1, sem1)
    reduce_into(rows1, A_CHUNK_ATOMS)
    issue(c0 + 3, rows1, sem1)
    drain(c0 + 2, rows0, sem0)
    reduce_into(rows0, 2 * A_CHUNK_ATOMS)

    @pl.when(c0 + 4 < A_CHUNKS)
    def _():
      issue(c0 + 4, rows0, sem0)

    drain(c0 + 3, rows1, sem1)
    reduce_into(rows1, 3 * A_CHUNK_ATOMS)

    @pl.when(c0 + 5 < A_CHUNKS)
    def _():
      issue(c0 + 5, rows1, sem1)

    pltpu.sync_copy(
        sum_v,
        out_hbm.at[pl.ds(base_atom + g * 4 * A_CHUNK_ATOMS,
                         4 * A_CHUNK_ATOMS)])
    return 0

  lax.fori_loop(0, nq, quad, 0)


@functools.partial(
    pl.kernel,
    out_type=jax.ShapeDtypeStruct((N_BONDS, HIDDEN), jnp.float32),
    mesh=_sc_mesh,
    scratch_types=[
        pltpu.VMEM((B_CHUNKS, B_CHUNK), jnp.int32),
        pltpu.VMEM((B_CHUNKS, B_CHUNK), jnp.int32),
        pltpu.VMEM((B_CHUNK, HIDDEN), jnp.float32),
        pltpu.VMEM((B_CHUNK, HIDDEN), jnp.float32),
        pltpu.VMEM((B_CHUNK, HIDDEN), jnp.float32),
        pltpu.VMEM((B_CHUNK, HIDDEN), jnp.float32),
        pltpu.SemaphoreType.DMA,
        pltpu.SemaphoreType.DMA,
        pltpu.SemaphoreType.DMA,
        pltpu.SemaphoreType.DMA,
    ],
)
def _sc_edge_msg(a_msg_hbm, msg_hbm, b2a_hbm, brev_hbm, h_hbm,
                 b2a_v, brev_v, am0, rev0, am1, rev1,
                 sem_a0, sem_b0, sem_a1, sem_b1):
  wid = _worker_id()
  base_e = wid * E_PER_W
  pltpu.sync_copy(b2a_hbm.at[wid], b2a_v)
  pltpu.sync_copy(brev_hbm.at[wid], brev_v)

  def issue(ci, am, rev, sa, sb):
    pltpu.async_copy(a_msg_hbm.at[b2a_v.at[ci]], am, sa)
    pltpu.async_copy(msg_hbm.at[brev_v.at[ci]], rev, sb)

  def drain(ci, am, rev, sa, sb):
    pltpu.make_async_copy(a_msg_hbm.at[b2a_v.at[ci]], am, sa).wait()
    pltpu.make_async_copy(msg_hbm.at[brev_v.at[ci]], rev, sb).wait()

  def sub_write(ci, am, rev):
    def sub(t, _):
      e = t // (HIDDEN // LANES) * 4
      col = (t % (HIDDEN // LANES)) * LANES
      for q in range(4):
        am[e + q, pl.ds(col, LANES)] = (
            am[e + q, pl.ds(col, LANES)] - rev[e + q, pl.ds(col, LANES)])
      return 0

    lax.fori_loop(0, B_CHUNK * (HIDDEN // LANES) // 4, sub, 0)
    pltpu.sync_copy(am, h_hbm.at[pl.ds(base_e + ci * B_CHUNK, B_CHUNK)])

  n2 = B_CHUNKS // 2
  issue(0, am0, rev0, sem_a0, sem_b0)

  def pair(c2, _):
    c0 = 2 * c2
    issue(c0 + 1, am1, rev1, sem_a1, sem_b1)
    drain(c0, am0, rev0, sem_a0, sem_b0)
    sub_write(c0, am0, rev0)
    issue(c0 + 2, am0, rev0, sem_a0, sem_b0)
    drain(c0 + 1, am1, rev1, sem_a1, sem_b1)
    sub_write(c0 + 1, am1, rev1)
    return 0

  lax.fori_loop(0, n2, pair, 0)
  last = 2 * n2
  drain(last, am0, rev0, sem_a0, sem_b0)
  sub_write(last, am0, rev0)


E_TILE = 1280


def _init_body(fb_ref, wi_ref, wih_ref, bih_ref, inp_ref, gi_ref):
  fb = fb_ref[...]
  inp = lax.dot_general(fb, wi_ref[...], (((1,), (1,)), ((), ())),
                        preferred_element_type=jnp.float32)
  inp_ref[...] = inp
  gi = lax.dot_general(inp, wih_ref[...], (((1,), (1,)), ((), ())),
                       preferred_element_type=jnp.float32) + bih_ref[...]
  gi_ref[...] = gi.astype(jnp.bfloat16)


def _tc_init(f_bonds, w_i, gru_w_ih, gru_b_ih):
  grid = (N_BONDS // E_TILE,)
  return pl.pallas_call(
      _init_body,
      grid=grid,
      in_specs=[
          pl.BlockSpec((E_TILE, BOND_FDIM), lambda i: (i, 0)),
          pl.BlockSpec((HIDDEN, BOND_FDIM), lambda i: (0, 0)),
          pl.BlockSpec((3 * HIDDEN, HIDDEN), lambda i: (0, 0)),
          pl.BlockSpec((1, 3 * HIDDEN), lambda i: (0, 0)),
      ],
      out_specs=[
          pl.BlockSpec((E_TILE, HIDDEN), lambda i: (i, 0)),
          pl.BlockSpec((E_TILE, 3 * HIDDEN), lambda i: (i, 0)),
      ],
      out_shape=[
          jax.ShapeDtypeStruct((N_BONDS, HIDDEN), jnp.float32),
          jax.ShapeDtypeStruct((N_BONDS, 3 * HIDDEN), jnp.bfloat16),
      ],
  )(f_bonds, w_i, gru_w_ih, gru_b_ih.reshape(1, -1))


def _gru_body(gi_ref, h_ref, whh_ref, bhh_ref, out_ref):
  h = h_ref[...]
  gh = lax.dot_general(h, whh_ref[...], (((1,), (1,)), ((), ())),
                       preferred_element_type=jnp.float32) + bhh_ref[...]
  gi = gi_ref[...].astype(jnp.float32)
  r = jax.nn.sigmoid(gi[:, :HIDDEN] + gh[:, :HIDDEN])
  z = jax.nn.sigmoid(gi[:, HIDDEN:2 * HIDDEN] + gh[:, HIDDEN:2 * HIDDEN])
  n = jnp.tanh(gi[:, 2 * HIDDEN:] + r * gh[:, 2 * HIDDEN:])
  m = (1.0 - z) * n + z * h
  row = lax.broadcasted_iota(jnp.int32, m.shape, 0)
  is_first = (pl.program_id(0) == 0)
  m = jnp.where(jnp.logical_and(row == 0, is_first), 0.0, m)
  out_ref[...] = m


def _tc_gru(gi, h, gru_w_hh, gru_b_hh):
  grid = (N_BONDS // E_TILE,)
  return pl.pallas_call(
      _gru_body,
      grid=grid,
      in_specs=[
          pl.BlockSpec((E_TILE, 3 * HIDDEN), lambda i: (i, 0)),
          pl.BlockSpec((E_TILE, HIDDEN), lambda i: (i, 0)),
          pl.BlockSpec((3 * HIDDEN, HIDDEN), lambda i: (0, 0)),
          pl.BlockSpec((1, 3 * HIDDEN), lambda i: (0, 0)),
      ],
      out_specs=pl.BlockSpec((E_TILE, HIDDEN), lambda i: (i, 0)),
      out_shape=jax.ShapeDtypeStruct((N_BONDS, HIDDEN), jnp.float32),
  )(gi, h, gru_w_hh, gru_b_hh.reshape(1, -1))


N_TILE = 1000


def _out_body(fa_ref, am_ref, wa_ref, wm_ref, b_ref, out_ref):
  acc = lax.dot_general(fa_ref[...], wa_ref[...], (((1,), (1,)), ((), ())),
                        preferred_element_type=jnp.float32)
  acc = acc + lax.dot_general(am_ref[...], wm_ref[...], (((1,), (1,)), ((), ())),
                              preferred_element_type=jnp.float32)
  out_ref[...] = jnp.maximum(acc + b_ref[...], 0.0)


def _tc_out(f_atoms, a_msg, W_o_w, W_o_b):
  grid = (N_ATOMS // N_TILE,)
  wa = W_o_w[:, :ATOM_FDIM]
  wm = W_o_w[:, ATOM_FDIM:]
  return pl.pallas_call(
      _out_body,
      grid=grid,
      in_specs=[
          pl.BlockSpec((N_TILE, ATOM_FDIM), lambda i: (i, 0)),
          pl.BlockSpec((N_TILE, HIDDEN), lambda i: (i, 0)),
          pl.BlockSpec((HIDDEN, ATOM_FDIM), lambda i: (0, 0)),
          pl.BlockSpec((HIDDEN, HIDDEN), lambda i: (0, 0)),
          pl.BlockSpec((1, HIDDEN), lambda i: (0, 0)),
      ],
      out_specs=pl.BlockSpec((N_TILE, HIDDEN), lambda i: (i, 0)),
      out_shape=jax.ShapeDtypeStruct((N_ATOMS, HIDDEN), jnp.float32),
  )(f_atoms, a_msg, wa, wm, W_o_b.reshape(1, -1))


def kernel(f_atoms, f_bonds, a2b, b2a, b2revb, undirected_b2a,
           w_i, gru_w_ih, gru_w_hh, gru_b_ih, gru_b_hh, W_o_w, W_o_b):
  pad_idx = jnp.arange(
      (NPAD - N_ATOMS) * MAX_NB, dtype=jnp.int32).reshape(
          NPAD - N_ATOMS, MAX_NB)
  a2b_pad = jnp.concatenate([a2b.astype(jnp.int32), pad_idx], axis=0)
  a2b_rs = a2b_pad.reshape(NW, A_CHUNKS, A_IDX_ROW)
  b2a_rs = b2a.astype(jnp.int32).reshape(NW, B_CHUNKS, B_CHUNK)
  brev_rs = b2revb.astype(jnp.int32).reshape(NW, B_CHUNKS, B_CHUNK)

  inp, gi = _tc_init(f_bonds, w_i, gru_w_ih, gru_b_ih)
  msg = inp
  for _ in range(DEPTH - 1):
    a_msg = _sc_gather_sum(msg, a2b_rs)
    h = _sc_edge_msg(a_msg, msg, b2a_rs, brev_rs)
    msg = _tc_gru(gi, h, gru_w_hh, gru_b_hh)
  a_msg = _sc_gather_sum(msg, a2b_rs)
  return _tc_out(f_atoms, a_msg[:N_ATOMS], W_o_w, W_o_b)


# speedup vs baseline: 1.7460x; 1.0378x over previous
import functools

import jax
import jax.numpy as jnp
from jax import lax
from jax.experimental import pallas as pl
from jax.experimental.pallas import tpu as pltpu
from jax.experimental.pallas import tpu_sc as plsc

N_ATOMS = 10000
N_BONDS = 320000
MAX_NB = 32
ATOM_FDIM = 128
BOND_FDIM = 144
HIDDEN = 256
DEPTH = 4

NC = 2
NS = 16
NW = NC * NS
LANES = 16
PLANES = 2 * LANES

AT_PER_W = 320
NPAD = NW * AT_PER_W
A_IDX_ROW = 128
A_CHUNK_ATOMS = A_IDX_ROW // MAX_NB
A_CHUNKS = AT_PER_W // A_CHUNK_ATOMS

E_PER_W = N_BONDS // NW
B_CHUNK = 80
B_CHUNKS = E_PER_W // B_CHUNK

_sc_mesh = plsc.VectorSubcoreMesh(
    core_axis_name="c", subcore_axis_name="s", num_cores=NC, num_subcores=NS)


def _worker_id():
  return lax.axis_index("s") * NC + lax.axis_index("c")


@functools.partial(
    pl.kernel,
    out_type=jax.ShapeDtypeStruct((NPAD, HIDDEN), jnp.float32),
    mesh=_sc_mesh,
    scratch_types=[
        pltpu.VMEM((A_CHUNKS, A_IDX_ROW), jnp.int32),
        pltpu.VMEM((A_IDX_ROW, HIDDEN), jnp.float32),
        pltpu.VMEM((A_IDX_ROW, HIDDEN), jnp.float32),
        pltpu.VMEM((4 * A_CHUNK_ATOMS, HIDDEN), jnp.float32),
        pltpu.SemaphoreType.DMA,
        pltpu.SemaphoreType.DMA,
    ],
)
def _sc_gather_sum(msg_hbm, a2b_hbm, out_hbm, idx_v, rows0, rows1, sum_v,
                   sem0, sem1):
  wid = _worker_id()
  base_atom = wid * AT_PER_W
  pltpu.sync_copy(a2b_hbm.at[wid], idx_v)

  def issue(ci, buf, sem):
    pltpu.async_copy(msg_hbm.at[idx_v.at[ci]], buf, sem)

  def drain(ci, buf, sem):
    pltpu.make_async_copy(msg_hbm.at[idx_v.at[ci]], buf, sem).wait()

  def reduce_into(buf, atom_off):
    def red(a, _):
      base = a * MAX_NB
      for j in range(HIDDEN // LANES):
        col = j * LANES
        vals = [buf[base + r, pl.ds(col, LANES)] for r in range(MAX_NB)]
        while len(vals) > 1:
          vals = [vals[i] + vals[i + 1] for i in range(0, len(vals), 2)]
        sum_v[atom_off + a, pl.ds(col, LANES)] = vals[0]
      return 0

    lax.fori_loop(0, A_CHUNK_ATOMS, red, 0)

  nq = A_CHUNKS // 4
  issue(0, rows0, sem0)
  issue(1, rows1, sem1)

  def quad(g, _):
    c0 = 4 * g
    drain(c0, rows0, sem0)
    reduce_into(rows0, 0)
    issue(c0 + 2, rows0, sem0)
    drain(c0 + 1, rows1, sem1)
    reduce_into(rows1, A_CHUNK_ATOMS)
    issue(c0 + 3, rows1, sem1)
    drain(c0 + 2, rows0, sem0)
    reduce_into(rows0, 2 * A_CHUNK_ATOMS)

    @pl.when(c0 + 4 < A_CHUNKS)
    def _():
      issue(c0 + 4, rows0, sem0)

    drain(c0 + 3, rows1, sem1)
    reduce_into(rows1, 3 * A_CHUNK_ATOMS)

    @pl.when(c0 + 5 < A_CHUNKS)
    def _():
      issue(c0 + 5, rows1, sem1)

    pltpu.sync_copy(
        sum_v,
        out_hbm.at[pl.ds(base_atom + g * 4 * A_CHUNK_ATOMS,
                         4 * A_CHUNK_ATOMS)])
    return 0

  lax.fori_loop(0, nq, quad, 0)


@functools.partial(
    pl.kernel,
    out_type=jax.ShapeDtypeStruct((N_BONDS, HIDDEN), jnp.float32),
    mesh=_sc_mesh,
    scratch_types=[
        pltpu.VMEM((B_CHUNKS, B_CHUNK), jnp.int32),
        pltpu.VMEM((B_CHUNKS, B_CHUNK), jnp.int32),
        pltpu.VMEM((B_CHUNK, HIDDEN), jnp.float32),
        pltpu.VMEM((B_CHUNK, HIDDEN), jnp.float32),
        pltpu.VMEM((B_CHUNK, HIDDEN), jnp.float32),
        pltpu.VMEM((B_CHUNK, HIDDEN), jnp.float32),
        pltpu.SemaphoreType.DMA,
        pltpu.SemaphoreType.DMA,
        pltpu.SemaphoreType.DMA,
        pltpu.SemaphoreType.DMA,
    ],
)
def _sc_edge_msg(a_msg_hbm, msg_hbm, b2a_hbm, brev_hbm, h_hbm,
                 b2a_v, brev_v, am0, rev0, am1, rev1,
                 sem_a0, sem_b0, sem_a1, sem_b1):
  wid = _worker_id()
  base_e = wid * E_PER_W
  pltpu.sync_copy(b2a_hbm.at[wid], b2a_v)
  pltpu.sync_copy(brev_hbm.at[wid], brev_v)

  def issue(ci, am, rev, sa, sb):
    pltpu.async_copy(a_msg_hbm.at[b2a_v.at[ci]], am, sa)
    pltpu.async_copy(msg_hbm.at[brev_v.at[ci]], rev, sb)

  def drain(ci, am, rev, sa, sb):
    pltpu.make_async_copy(a_msg_hbm.at[b2a_v.at[ci]], am, sa).wait()
    pltpu.make_async_copy(msg_hbm.at[brev_v.at[ci]], rev, sb).wait()

  def sub_write(ci, am, rev):
    def sub(t, _):
      e = t // (HIDDEN // LANES) * 4
      col = (t % (HIDDEN // LANES)) * LANES
      for q in range(4):
        am[e + q, pl.ds(col, LANES)] = (
            am[e + q, pl.ds(col, LANES)] - rev[e + q, pl.ds(col, LANES)])
      return 0

    lax.fori_loop(0, B_CHUNK * (HIDDEN // LANES) // 4, sub, 0)
    pltpu.sync_copy(am, h_hbm.at[pl.ds(base_e + ci * B_CHUNK, B_CHUNK)])

  n2 = B_CHUNKS // 2
  issue(0, am0, rev0, sem_a0, sem_b0)

  def pair(c2, _):
    c0 = 2 * c2
    issue(c0 + 1, am1, rev1, sem_a1, sem_b1)
    drain(c0, am0, rev0, sem_a0, sem_b0)
    sub_write(c0, am0, rev0)
    issue(c0 + 2, am0, rev0, sem_a0, sem_b0)
    drain(c0 + 1, am1, rev1, sem_a1, sem_b1)
    sub_write(c0 + 1, am1, rev1)
    return 0

  lax.fori_loop(0, n2, pair, 0)
  last = 2 * n2
  drain(last, am0, rev0, sem_a0, sem_b0)
  sub_write(last, am0, rev0)


E_TILE = 1600


def _init_body(fb_ref, wi_ref, wih_ref, bih_ref, inp_ref, gi_ref):
  fb = fb_ref[...].astype(jnp.bfloat16)
  inp = lax.dot_general(fb, wi_ref[...], (((1,), (1,)), ((), ())),
                        preferred_element_type=jnp.float32)
  inp_ref[...] = inp
  gi = lax.dot_general(inp.astype(jnp.bfloat16), wih_ref[...],
                       (((1,), (1,)), ((), ())),
                       preferred_element_type=jnp.float32) + bih_ref[...]
  gi_ref[...] = gi.astype(jnp.bfloat16)


def _tc_init(f_bonds, w_i, gru_w_ih, gru_b_ih):
  grid = (N_BONDS // E_TILE,)
  return pl.pallas_call(
      _init_body,
      grid=grid,
      in_specs=[
          pl.BlockSpec((E_TILE, BOND_FDIM), lambda i: (i, 0)),
          pl.BlockSpec((HIDDEN, BOND_FDIM), lambda i: (0, 0)),
          pl.BlockSpec((3 * HIDDEN, HIDDEN), lambda i: (0, 0)),
          pl.BlockSpec((1, 3 * HIDDEN), lambda i: (0, 0)),
      ],
      out_specs=[
          pl.BlockSpec((E_TILE, HIDDEN), lambda i: (i, 0)),
          pl.BlockSpec((E_TILE, 3 * HIDDEN), lambda i: (i, 0)),
      ],
      out_shape=[
          jax.ShapeDtypeStruct((N_BONDS, HIDDEN), jnp.float32),
          jax.ShapeDtypeStruct((N_BONDS, 3 * HIDDEN), jnp.bfloat16),
      ],
  )(f_bonds, w_i.astype(jnp.bfloat16),
    gru_w_ih.astype(jnp.bfloat16), gru_b_ih.reshape(1, -1))


def _gru_body(gi_ref, h_ref, whh_ref, bhh_ref, out_ref):
  h = h_ref[...]
  gh = lax.dot_general(h.astype(jnp.bfloat16), whh_ref[...],
                       (((1,), (1,)), ((), ())),
                       preferred_element_type=jnp.float32) + bhh_ref[...]
  gi = gi_ref[...].astype(jnp.float32)
  r = jax.nn.sigmoid(gi[:, :HIDDEN] + gh[:, :HIDDEN])
  z = jax.nn.sigmoid(gi[:, HIDDEN:2 * HIDDEN] + gh[:, HIDDEN:2 * HIDDEN])
  n = jnp.tanh(gi[:, 2 * HIDDEN:] + r * gh[:, 2 * HIDDEN:])
  m = (1.0 - z) * n + z * h
  row = lax.broadcasted_iota(jnp.int32, m.shape, 0)
  is_first = (pl.program_id(0) == 0)
  m = jnp.where(jnp.logical_and(row == 0, is_first), 0.0, m)
  out_ref[...] = m


def _tc_gru(gi, h, gru_w_hh, gru_b_hh):
  grid = (N_BONDS // E_TILE,)
  return pl.pallas_call(
      _gru_body,
      grid=grid,
      in_specs=[
          pl.BlockSpec((E_TILE, 3 * HIDDEN), lambda i: (i, 0)),
          pl.BlockSpec((E_TILE, HIDDEN), lambda i: (i, 0)),
          pl.BlockSpec((3 * HIDDEN, HIDDEN), lambda i: (0, 0)),
          pl.BlockSpec((1, 3 * HIDDEN), lambda i: (0, 0)),
      ],
      out_specs=pl.BlockSpec((E_TILE, HIDDEN), lambda i: (i, 0)),
      out_shape=jax.ShapeDtypeStruct((N_BONDS, HIDDEN), jnp.float32),
  )(gi, h, gru_w_hh.astype(jnp.bfloat16),
    gru_b_hh.reshape(1, -1))


N_TILE = 1000


def _out_body(fa_ref, am_ref, wa_ref, wm_ref, b_ref, out_ref):
  acc = lax.dot_general(fa_ref[...], wa_ref[...], (((1,), (1,)), ((), ())),
                        preferred_element_type=jnp.float32)
  acc = acc + lax.dot_general(am_ref[...], wm_ref[...], (((1,), (1,)), ((), ())),
                              preferred_element_type=jnp.float32)
  out_ref[...] = jnp.maximum(acc + b_ref[...], 0.0)


def _tc_out(f_atoms, a_msg, W_o_w, W_o_b):
  grid = (N_ATOMS // N_TILE,)
  wa = W_o_w[:, :ATOM_FDIM]
  wm = W_o_w[:, ATOM_FDIM:]
  return pl.pallas_call(
      _out_body,
      grid=grid,
      in_specs=[
          pl.BlockSpec((N_TILE, ATOM_FDIM), lambda i: (i, 0)),
          pl.BlockSpec((N_TILE, HIDDEN), lambda i: (i, 0)),
          pl.BlockSpec((HIDDEN, ATOM_FDIM), lambda i: (0, 0)),
          pl.BlockSpec((HIDDEN, HIDDEN), lambda i: (0, 0)),
          pl.BlockSpec((1, HIDDEN), lambda i: (0, 0)),
      ],
      out_specs=pl.BlockSpec((N_TILE, HIDDEN), lambda i: (i, 0)),
      out_shape=jax.ShapeDtypeStruct((N_ATOMS, HIDDEN), jnp.float32),
  )(f_atoms, a_msg, wa, wm, W_o_b.reshape(1, -1))


def kernel(f_atoms, f_bonds, a2b, b2a, b2revb, undirected_b2a,
           w_i, gru_w_ih, gru_w_hh, gru_b_ih, gru_b_hh, W_o_w, W_o_b):
  pad_idx = jnp.arange(
      (NPAD - N_ATOMS) * MAX_NB, dtype=jnp.int32).reshape(
          NPAD - N_ATOMS, MAX_NB)
  a2b_pad = jnp.concatenate([a2b.astype(jnp.int32), pad_idx], axis=0)
  a2b_rs = a2b_pad.reshape(NW, A_CHUNKS, A_IDX_ROW)
  b2a_rs = b2a.astype(jnp.int32).reshape(NW, B_CHUNKS, B_CHUNK)
  brev_rs = b2revb.astype(jnp.int32).reshape(NW, B_CHUNKS, B_CHUNK)

  inp, gi = _tc_init(f_bonds, w_i, gru_w_ih, gru_b_ih)
  msg = inp
  for _ in range(DEPTH - 1):
    a_msg = _sc_gather_sum(msg, a2b_rs)
    h = _sc_edge_msg(a_msg, msg, b2a_rs, brev_rs)
    msg = _tc_gru(gi, h, gru_w_hh, gru_b_hh)
  a_msg = _sc_gather_sum(msg, a2b_rs)
  return _tc_out(f_atoms, a_msg[:N_ATOMS], W_o_w, W_o_b)
